# baseline (speedup 1.0000x reference)
import functools

import jax
import jax.numpy as jnp
from jax import lax
from jax.experimental import pallas as pl
from jax.experimental.pallas import tpu as pltpu
from jax.experimental.pallas import tpu_sc as plsc

_K = 16
_RB = 512
_RC = 256
_RF = 512
_SC_CORES = 2
_SC_SUBCORES = 16
_GCHUNK = 1024


def _mish(v):
    sp = jnp.maximum(v, 0.0) + jnp.log1p(jnp.exp(-jnp.abs(v)))
    return v * jnp.tanh(sp)


def _topk_body(ps_ref, pos_ref, idx_ref, ds_ref):
    r = ps_ref.shape[1]
    n = pos_ref.shape[2]
    xb = ps_ref[0, :, 0:1]
    yb = ps_ref[0, :, 1:2]
    zb = ps_ref[0, :, 2:3]
    xf = pos_ref[0, 0:1, :]
    yf = pos_ref[0, 1:2, :]
    zf = pos_ref[0, 2:3, :]
    sqb = xb * xb + yb * yb + zb * zb
    sqf = xf * xf + yf * yf + zf * zf
    dot = jnp.dot(ps_ref[0].astype(jnp.bfloat16),
                  pos_ref[0].astype(jnp.bfloat16),
                  preferred_element_type=jnp.float32)
    d = sqb + sqf - 2.0 * dot
    cols = lax.broadcasted_iota(jnp.int32, (r, n), 1)
    cur = d
    picks = []
    for _ in range(_K):
        j = jnp.argmin(cur, axis=1, keepdims=True).astype(jnp.int32)
        picks.append(j)
        cur = jnp.where(cols == j, jnp.float32(jnp.inf), cur)
    idx_ref[0] = jnp.concatenate(picks, axis=1)
    ds_ref[0] = jnp.sum(jnp.where(jnp.isinf(cur), d, 0.0), axis=1,
                        keepdims=True)


def _topk(pos):
    b, _, n = pos.shape
    ps = jnp.transpose(pos, (0, 2, 1))
    rb = min(_RB, n)
    nb = n // rb
    idx, ds = pl.pallas_call(
        _topk_body,
        grid=(b, nb),
        in_specs=[
            pl.BlockSpec((1, rb, 3), lambda bi, i: (bi, i, 0)),
            pl.BlockSpec((1, 3, n), lambda bi, i: (bi, 0, 0)),
        ],
        out_specs=[
            pl.BlockSpec((1, rb, _K), lambda bi, i: (bi, i, 0)),
            pl.BlockSpec((1, rb, 1), lambda bi, i: (bi, i, 0)),
        ],
        out_shape=[
            jax.ShapeDtypeStruct((b, n, _K), jnp.int32),
            jax.ShapeDtypeStruct((b, n, 1), jnp.float32),
        ],
    )(ps, pos)
    return idx, ds[..., 0]


def _gather_sc(table, idx_flat):
    tot = idx_flat.shape[0]
    c = table.shape[1]
    nw = _SC_CORES * _SC_SUBCORES
    per_w = tot // nw
    chunk = min(_GCHUNK, per_w)
    nch = per_w // chunk
    mesh = plsc.VectorSubcoreMesh(
        core_axis_name="c", subcore_axis_name="s",
        num_cores=_SC_CORES, num_subcores=_SC_SUBCORES)

    @functools.partial(
        pl.kernel,
        out_type=jax.ShapeDtypeStruct((tot, c), jnp.float32),
        mesh=mesh,
        scratch_types=[
            pltpu.VMEM((chunk,), jnp.int32),
            pltpu.VMEM((chunk, c), jnp.float32),
            pltpu.SemaphoreType.DMA,
        ],
        compiler_params=pltpu.CompilerParams(use_tc_tiling_on_sc=False),
    )
    def k(table_hbm, idx_hbm, out_hbm, idx_v, rows_v, sem):
        wid = lax.axis_index("s") * _SC_CORES + lax.axis_index("c")
        base = wid * per_w
        for i in range(nch):
            off = base + i * chunk
            pltpu.sync_copy(idx_hbm.at[pl.ds(off, chunk)], idx_v)
            pltpu.async_copy(table_hbm.at[idx_v], rows_v, sem).wait()
            pltpu.sync_copy(rows_v, out_hbm.at[pl.ds(off, chunk)])

    return k(table, idx_flat)


def _conv1_body(nbr_ref, xt_ref, w1a_ref, w1b_ref, b1_ref,
                h1_ref, s1_ref, q1_ref):
    r = xt_ref.shape[0]
    c = xt_ref.shape[1]
    nbr = nbr_ref[...].reshape(r, _K, c)
    xi = xt_ref[...]
    en = (nbr - xi[:, None, :]).reshape(r * _K, c)
    ha = jnp.dot(en, w1a_ref[...], preferred_element_type=jnp.float32)
    hb = jnp.dot(xi, w1b_ref[...], preferred_element_type=jnp.float32)
    h = (ha.reshape(r, _K, -1) + hb[:, None, :]).reshape(r * _K, -1)
    h = h + b1_ref[...]
    h1_ref[...] = h
    s = jnp.sum(h, axis=0, keepdims=True)
    q = jnp.sum(h * h, axis=0, keepdims=True)

    @pl.when(pl.program_id(0) == 0)
    def _():
        s1_ref[...] = jnp.zeros_like(s1_ref)
        q1_ref[...] = jnp.zeros_like(q1_ref)

    s1_ref[...] += s
    q1_ref[...] += q


def _conv1(nbr, xt, w1, b1):
    rows, c = xt.shape
    dmid = w1.shape[0]
    rc = min(_RC, rows)
    ng = rows // rc
    w1a = jnp.transpose(w1[:, :c])
    w1b = jnp.transpose(w1[:, c:])
    return pl.pallas_call(
        _conv1_body,
        grid=(ng,),
        in_specs=[
            pl.BlockSpec((rc * _K, c), lambda g: (g, 0)),
            pl.BlockSpec((rc, c), lambda g: (g, 0)),
            pl.BlockSpec((c, dmid), lambda g: (0, 0)),
            pl.BlockSpec((c, dmid), lambda g: (0, 0)),
            pl.BlockSpec((1, dmid), lambda g: (0, 0)),
        ],
        out_specs=[
            pl.BlockSpec((rc * _K, dmid), lambda g: (g, 0)),
            pl.BlockSpec((1, dmid), lambda g: (0, 0)),
            pl.BlockSpec((1, dmid), lambda g: (0, 0)),
        ],
        out_shape=[
            jax.ShapeDtypeStruct((rows * _K, dmid), jnp.float32),
            jax.ShapeDtypeStruct((1, dmid), jnp.float32),
            jax.ShapeDtypeStruct((1, dmid), jnp.float32),
        ],
    )(nbr, xt, w1a, w1b, b1.reshape(1, -1))


def _conv2_body(h1_ref, sc_ref, sh_ref, w2_ref, b2_ref,
                h2_ref, s2_ref, q2_ref):
    t = h1_ref[...] * sc_ref[...] + sh_ref[...]
    g = _mish(t)
    h = jnp.dot(g, w2_ref[...], preferred_element_type=jnp.float32) + b2_ref[...]
    h2_ref[...] = h
    s = jnp.sum(h, axis=0, keepdims=True)
    q = jnp.sum(h * h, axis=0, keepdims=True)

    @pl.when(pl.program_id(0) == 0)
    def _():
        s2_ref[...] = jnp.zeros_like(s2_ref)
        q2_ref[...] = jnp.zeros_like(q2_ref)

    s2_ref[...] += s
    q2_ref[...] += q


def _conv2(h1, sc1, sh1, w2, b2):
    rows_k, dmid = h1.shape
    dout = w2.shape[0]
    blk = min(_RC * _K, rows_k)
    ng = rows_k // blk
    w2t = jnp.transpose(w2)
    return pl.pallas_call(
        _conv2_body,
        grid=(ng,),
        in_specs=[
            pl.BlockSpec((blk, dmid), lambda g: (g, 0)),
            pl.BlockSpec((1, dmid), lambda g: (0, 0)),
            pl.BlockSpec((1, dmid), lambda g: (0, 0)),
            pl.BlockSpec((dmid, dout), lambda g: (0, 0)),
            pl.BlockSpec((1, dout), lambda g: (0, 0)),
        ],
        out_specs=[
            pl.BlockSpec((blk, dout), lambda g: (g, 0)),
            pl.BlockSpec((1, dout), lambda g: (0, 0)),
            pl.BlockSpec((1, dout), lambda g: (0, 0)),
        ],
        out_shape=[
            jax.ShapeDtypeStruct((rows_k, dout), jnp.float32),
            jax.ShapeDtypeStruct((1, dout), jnp.float32),
            jax.ShapeDtypeStruct((1, dout), jnp.float32),
        ],
    )(h1, sc1, sh1, w2t, b2.reshape(1, -1))


def _final_body(h2_ref, sc_ref, sh_ref, r_ref, s3_ref):
    r = r_ref.shape[0]
    t = h2_ref[...] * sc_ref[...] + sh_ref[...]
    m = _mish(t)
    r_ref[...] = jnp.max(m.reshape(r, _K, -1), axis=1)
    s = jnp.sum(m, axis=0, keepdims=True)

    @pl.when(pl.program_id(0) == 0)
    def _():
        s3_ref[...] = jnp.zeros_like(s3_ref)

    s3_ref[...] += s


def _finalize(h2, sc2, sh2, n):
    rows_k, dout = h2.shape
    rc = min(_RC, n)
    nb = n // rc
    return pl.pallas_call(
        _final_body,
        grid=(nb,),
        in_specs=[
            pl.BlockSpec((rc * _K, dout), lambda i: (i, 0)),
            pl.BlockSpec((1, dout), lambda i: (0, 0)),
            pl.BlockSpec((1, dout), lambda i: (0, 0)),
        ],
        out_specs=[
            pl.BlockSpec((rc, dout), lambda i: (i, 0)),
            pl.BlockSpec((1, dout), lambda i: (0, 0)),
        ],
        out_shape=[
            jax.ShapeDtypeStruct((n, dout), jnp.float32),
            jax.ShapeDtypeStruct((1, dout), jnp.float32),
        ],
    )(h2, sc2, sh2)


def _scale_body(cnt_inv, r_ref, s3_ref, w1_ref, b1_ref, w2t_ref, b2_ref,
                out_ref):
    sm = s3_ref[...] * cnt_inv
    z = jnp.sum(w1_ref[...] * sm, axis=1, keepdims=True)
    z = jnp.maximum(z + b1_ref[...], 0.0)
    e = jnp.sum(w2t_ref[...] * z, axis=0, keepdims=True)
    e = 1.0 / (1.0 + jnp.exp(-(e + b2_ref[...])))
    h = r_ref[...] * e
    out_ref[...] = jnp.transpose(h)


def _scale_out(r, s3, se_w1, se_b1, se_w2, se_b2, n, cnt):
    dout = r.shape[1]
    dse = se_w1.shape[0]
    rf = min(_RF, n)
    nb = n // rf
    body = functools.partial(_scale_body, 1.0 / cnt)
    return pl.pallas_call(
        body,
        grid=(nb,),
        in_specs=[
            pl.BlockSpec((rf, dout), lambda i: (i, 0)),
            pl.BlockSpec((1, dout), lambda i: (0, 0)),
            pl.BlockSpec((dse, dout), lambda i: (0, 0)),
            pl.BlockSpec((dse, 1), lambda i: (0, 0)),
            pl.BlockSpec((dse, dout), lambda i: (0, 0)),
            pl.BlockSpec((1, dout), lambda i: (0, 0)),
        ],
        out_specs=pl.BlockSpec((dout, rf), lambda i: (0, i)),
        out_shape=jax.ShapeDtypeStruct((dout, n), jnp.float32),
    )(r, s3, se_w1, se_b1.reshape(-1, 1), jnp.transpose(se_w2), se_b2.reshape(1, -1))


def kernel(x, pos, conv1_W, conv1_b, bn1_g, bn1_b, conv2_W, conv2_b,
           bn2_g, bn2_b, se_W1, se_b1, se_W2, se_b2):
    b, c, n = x.shape
    cnt = b * n * _K

    xt = jnp.transpose(x, (0, 2, 1))

    idxs, dss, nbrs = [], [], []
    for bi in range(b):
        idx_b, ds_b = _topk(pos[bi:bi + 1])
        idxs.append(idx_b)
        dss.append(ds_b)
        nbrs.append(_gather_sc(xt[bi], idx_b.reshape(-1)))
    dist_sum = jnp.concatenate(dss, axis=0)

    c1 = [_conv1(nbrs[bi], xt[bi], conv1_W, conv1_b) for bi in range(b)]
    s1 = sum(t[1] for t in c1)
    q1 = sum(t[2] for t in c1)
    mu1 = s1 / cnt
    var1 = q1 / cnt - mu1 * mu1
    sc1 = bn1_g.reshape(1, -1) / jnp.sqrt(var1 + 1e-5)
    sh1 = bn1_b.reshape(1, -1) - mu1 * sc1

    c2 = [_conv2(c1[bi][0], sc1, sh1, conv2_W, conv2_b) for bi in range(b)]
    s2 = sum(t[1] for t in c2)
    q2 = sum(t[2] for t in c2)
    mu2 = s2 / cnt
    var2 = q2 / cnt - mu2 * mu2
    sc2 = bn2_g.reshape(1, -1) / jnp.sqrt(var2 + 1e-5)
    sh2 = bn2_b.reshape(1, -1) - mu2 * sc2

    fin = [_finalize(c2[bi][0], sc2, sh2, n) for bi in range(b)]
    res = [_scale_out(fin[bi][0], fin[bi][1], se_W1, se_b1, se_W2, se_b2,
                      n, n * _K) for bi in range(b)]
    residual = jnp.stack(res, axis=0)
    return residual, dist_sum


# speedup vs baseline: 1.0342x; 1.0342x over previous
import functools

import jax
import jax.numpy as jnp
from jax import lax
from jax.experimental import pallas as pl
from jax.experimental.pallas import tpu as pltpu
from jax.experimental.pallas import tpu_sc as plsc

_K = 16
_RB = 512
_RC = 256
_RF = 512
_SC_CORES = 2
_SC_SUBCORES = 16
_GCHUNK = 1024


def _mish(v):
    sp = jnp.maximum(v, 0.0) + jnp.log1p(jnp.exp(-jnp.abs(v)))
    return v * jnp.tanh(sp)


def _topk_body(ps_ref, pos_ref, idx_ref, ds_ref):
    r = ps_ref.shape[1]
    n = pos_ref.shape[2]
    xb = ps_ref[0, :, 0:1]
    yb = ps_ref[0, :, 1:2]
    zb = ps_ref[0, :, 2:3]
    xf = pos_ref[0, 0:1, :]
    yf = pos_ref[0, 1:2, :]
    zf = pos_ref[0, 2:3, :]
    sqb = xb * xb + yb * yb + zb * zb
    sqf = xf * xf + yf * yf + zf * zf
    dot = jnp.dot(ps_ref[0].astype(jnp.bfloat16),
                  pos_ref[0].astype(jnp.bfloat16),
                  preferred_element_type=jnp.float32)
    d = sqb + sqf - 2.0 * dot
    cols = lax.broadcasted_iota(jnp.int32, (r, n), 1)
    cur = d
    picks = []
    for _ in range(_K):
        j = jnp.argmin(cur, axis=1, keepdims=True).astype(jnp.int32)
        picks.append(j)
        cur = jnp.where(cols == j, jnp.float32(jnp.inf), cur)
    idx_ref[0] = jnp.concatenate(picks, axis=1)
    ds_ref[0] = jnp.sum(jnp.where(jnp.isinf(cur), d, 0.0), axis=1,
                        keepdims=True)


def _topk(pos):
    b, _, n = pos.shape
    ps = jnp.transpose(pos, (0, 2, 1))
    rb = min(_RB, n)
    nb = n // rb
    idx, ds = pl.pallas_call(
        _topk_body,
        grid=(b, nb),
        in_specs=[
            pl.BlockSpec((1, rb, 3), lambda bi, i: (bi, i, 0)),
            pl.BlockSpec((1, 3, n), lambda bi, i: (bi, 0, 0)),
        ],
        out_specs=[
            pl.BlockSpec((1, rb, _K), lambda bi, i: (bi, i, 0)),
            pl.BlockSpec((1, rb, 1), lambda bi, i: (bi, i, 0)),
        ],
        out_shape=[
            jax.ShapeDtypeStruct((b, n, _K), jnp.int32),
            jax.ShapeDtypeStruct((b, n, 1), jnp.float32),
        ],
    )(ps, pos)
    return idx, ds[..., 0]


def _gather_sc(table, idx_flat):
    tot = idx_flat.shape[0]
    c = table.shape[1]
    nw = _SC_CORES * _SC_SUBCORES
    per_w = tot // nw
    chunk = min(_GCHUNK, per_w)
    nch = per_w // chunk
    mesh = plsc.VectorSubcoreMesh(
        core_axis_name="c", subcore_axis_name="s",
        num_cores=_SC_CORES, num_subcores=_SC_SUBCORES)

    @functools.partial(
        pl.kernel,
        out_type=jax.ShapeDtypeStruct((tot, c), jnp.float32),
        mesh=mesh,
        scratch_types=[
            pltpu.VMEM((chunk,), jnp.int32),
            pltpu.VMEM((chunk, c), jnp.float32),
            pltpu.SemaphoreType.DMA,
        ],
        compiler_params=pltpu.CompilerParams(use_tc_tiling_on_sc=False),
    )
    def k(table_hbm, idx_hbm, out_hbm, idx_v, rows_v, sem):
        wid = lax.axis_index("s") * _SC_CORES + lax.axis_index("c")
        base = wid * per_w
        for i in range(nch):
            off = base + i * chunk
            pltpu.sync_copy(idx_hbm.at[pl.ds(off, chunk)], idx_v)
            pltpu.async_copy(table_hbm.at[idx_v], rows_v, sem).wait()
            pltpu.sync_copy(rows_v, out_hbm.at[pl.ds(off, chunk)])

    return k(table, idx_flat)


def _conv1_body(nbr_ref, xt_ref, w1a_ref, w1b_ref, b1_ref,
                h1_ref, s1_ref, q1_ref):
    r = xt_ref.shape[0]
    c = xt_ref.shape[1]
    nbr = nbr_ref[...].reshape(r, _K, c)
    xi = xt_ref[...]
    en = (nbr - xi[:, None, :]).reshape(r * _K, c)
    ha = jnp.dot(en, w1a_ref[...], preferred_element_type=jnp.float32)
    hb = jnp.dot(xi, w1b_ref[...], preferred_element_type=jnp.float32)
    h = (ha.reshape(r, _K, -1) + hb[:, None, :]).reshape(r * _K, -1)
    h = h + b1_ref[...]
    h1_ref[...] = h
    s = jnp.sum(h, axis=0, keepdims=True)
    q = jnp.sum(h * h, axis=0, keepdims=True)

    @pl.when(pl.program_id(0) == 0)
    def _():
        s1_ref[...] = jnp.zeros_like(s1_ref)
        q1_ref[...] = jnp.zeros_like(q1_ref)

    s1_ref[...] += s
    q1_ref[...] += q


def _conv1(nbr, xt, w1, b1):
    rows, c = xt.shape
    dmid = w1.shape[0]
    rc = min(_RC, rows)
    ng = rows // rc
    w1a = jnp.transpose(w1[:, :c])
    w1b = jnp.transpose(w1[:, c:])
    return pl.pallas_call(
        _conv1_body,
        grid=(ng,),
        in_specs=[
            pl.BlockSpec((rc * _K, c), lambda g: (g, 0)),
            pl.BlockSpec((rc, c), lambda g: (g, 0)),
            pl.BlockSpec((c, dmid), lambda g: (0, 0)),
            pl.BlockSpec((c, dmid), lambda g: (0, 0)),
            pl.BlockSpec((1, dmid), lambda g: (0, 0)),
        ],
        out_specs=[
            pl.BlockSpec((rc * _K, dmid), lambda g: (g, 0)),
            pl.BlockSpec((1, dmid), lambda g: (0, 0)),
            pl.BlockSpec((1, dmid), lambda g: (0, 0)),
        ],
        out_shape=[
            jax.ShapeDtypeStruct((rows * _K, dmid), jnp.float32),
            jax.ShapeDtypeStruct((1, dmid), jnp.float32),
            jax.ShapeDtypeStruct((1, dmid), jnp.float32),
        ],
    )(nbr, xt, w1a, w1b, b1.reshape(1, -1))


def _conv2_body(h1_ref, sc_ref, sh_ref, w2_ref, b2_ref,
                h2_ref, s2_ref, q2_ref):
    t = h1_ref[...] * sc_ref[...] + sh_ref[...]
    g = _mish(t)
    h = jnp.dot(g, w2_ref[...], preferred_element_type=jnp.float32) + b2_ref[...]
    h2_ref[...] = h
    s = jnp.sum(h, axis=0, keepdims=True)
    q = jnp.sum(h * h, axis=0, keepdims=True)

    @pl.when(pl.program_id(0) == 0)
    def _():
        s2_ref[...] = jnp.zeros_like(s2_ref)
        q2_ref[...] = jnp.zeros_like(q2_ref)

    s2_ref[...] += s
    q2_ref[...] += q


def _conv2(h1, sc1, sh1, w2, b2):
    rows_k, dmid = h1.shape
    dout = w2.shape[0]
    blk = min(_RC * _K, rows_k)
    ng = rows_k // blk
    w2t = jnp.transpose(w2)
    return pl.pallas_call(
        _conv2_body,
        grid=(ng,),
        in_specs=[
            pl.BlockSpec((blk, dmid), lambda g: (g, 0)),
            pl.BlockSpec((1, dmid), lambda g: (0, 0)),
            pl.BlockSpec((1, dmid), lambda g: (0, 0)),
            pl.BlockSpec((dmid, dout), lambda g: (0, 0)),
            pl.BlockSpec((1, dout), lambda g: (0, 0)),
        ],
        out_specs=[
            pl.BlockSpec((blk, dout), lambda g: (g, 0)),
            pl.BlockSpec((1, dout), lambda g: (0, 0)),
            pl.BlockSpec((1, dout), lambda g: (0, 0)),
        ],
        out_shape=[
            jax.ShapeDtypeStruct((rows_k, dout), jnp.float32),
            jax.ShapeDtypeStruct((1, dout), jnp.float32),
            jax.ShapeDtypeStruct((1, dout), jnp.float32),
        ],
    )(h1, sc1, sh1, w2t, b2.reshape(1, -1))


def _final_body(h2_ref, sc_ref, sh_ref, r_ref, s3_ref):
    r = r_ref.shape[0]
    t = h2_ref[...] * sc_ref[...] + sh_ref[...]
    m = _mish(t)
    r_ref[...] = jnp.max(m.reshape(r, _K, -1), axis=1)
    s = jnp.sum(m, axis=0, keepdims=True)

    @pl.when(pl.program_id(1) == 0)
    def _():
        s3_ref[...] = jnp.zeros_like(s3_ref)

    s3_ref[0] += s


def _finalize(h2, sc2, sh2, b, n):
    rows_k, dout = h2.shape
    rc = min(_RC, n)
    nb = n // rc
    return pl.pallas_call(
        _final_body,
        grid=(b, nb),
        in_specs=[
            pl.BlockSpec((rc * _K, dout), lambda bi, i: (bi * nb + i, 0)),
            pl.BlockSpec((1, dout), lambda bi, i: (0, 0)),
            pl.BlockSpec((1, dout), lambda bi, i: (0, 0)),
        ],
        out_specs=[
            pl.BlockSpec((rc, dout), lambda bi, i: (bi * nb + i, 0)),
            pl.BlockSpec((1, 1, dout), lambda bi, i: (bi, 0, 0)),
        ],
        out_shape=[
            jax.ShapeDtypeStruct((b * n, dout), jnp.float32),
            jax.ShapeDtypeStruct((b, 1, dout), jnp.float32),
        ],
    )(h2, sc2, sh2)


def _scale_body(cnt_inv, r_ref, s3_ref, w1_ref, b1_ref, w2t_ref, b2_ref,
                out_ref):
    sm = s3_ref[0] * cnt_inv
    z = jnp.sum(w1_ref[...] * sm, axis=1, keepdims=True)
    z = jnp.maximum(z + b1_ref[...], 0.0)
    e = jnp.sum(w2t_ref[...] * z, axis=0, keepdims=True)
    e = 1.0 / (1.0 + jnp.exp(-(e + b2_ref[...])))
    h = r_ref[...] * e
    out_ref[0] = jnp.transpose(h)


def _scale_out(r, s3, se_w1, se_b1, se_w2, se_b2, b, n, cnt):
    dout = r.shape[1]
    dse = se_w1.shape[0]
    rf = min(_RF, n)
    nb = n // rf
    body = functools.partial(_scale_body, 1.0 / cnt)
    return pl.pallas_call(
        body,
        grid=(b, nb),
        in_specs=[
            pl.BlockSpec((rf, dout), lambda bi, i: (bi * nb + i, 0)),
            pl.BlockSpec((1, 1, dout), lambda bi, i: (bi, 0, 0)),
            pl.BlockSpec((dse, dout), lambda bi, i: (0, 0)),
            pl.BlockSpec((dse, 1), lambda bi, i: (0, 0)),
            pl.BlockSpec((dse, dout), lambda bi, i: (0, 0)),
            pl.BlockSpec((1, dout), lambda bi, i: (0, 0)),
        ],
        out_specs=pl.BlockSpec((1, dout, rf), lambda bi, i: (bi, 0, i)),
        out_shape=jax.ShapeDtypeStruct((b, dout, n), jnp.float32),
    )(r, s3, se_w1, se_b1.reshape(-1, 1), jnp.transpose(se_w2), se_b2.reshape(1, -1))


def kernel(x, pos, conv1_W, conv1_b, bn1_g, bn1_b, conv2_W, conv2_b,
           bn2_g, bn2_b, se_W1, se_b1, se_W2, se_b2):
    b, c, n = x.shape
    cnt = b * n * _K

    idx, dist_sum = _topk(pos)

    xt = jnp.transpose(x, (0, 2, 1)).reshape(b * n, c)
    offs = (jnp.arange(b, dtype=jnp.int32) * n)[:, None, None]
    idx_flat = (idx + offs).reshape(-1)
    nbr = _gather_sc(xt, idx_flat)

    h1, s1, q1 = _conv1(nbr, xt, conv1_W, conv1_b)
    mu1 = s1 / cnt
    var1 = q1 / cnt - mu1 * mu1
    sc1 = bn1_g.reshape(1, -1) / jnp.sqrt(var1 + 1e-5)
    sh1 = bn1_b.reshape(1, -1) - mu1 * sc1

    h2, s2, q2 = _conv2(h1, sc1, sh1, conv2_W, conv2_b)
    mu2 = s2 / cnt
    var2 = q2 / cnt - mu2 * mu2
    sc2 = bn2_g.reshape(1, -1) / jnp.sqrt(var2 + 1e-5)
    sh2 = bn2_b.reshape(1, -1) - mu2 * sc2

    r, s3 = _finalize(h2, sc2, sh2, b, n)
    residual = _scale_out(r, s3, se_W1, se_b1, se_W2, se_b2, b, n, n * _K)
    return residual, dist_sum


# speedup vs baseline: 1.0538x; 1.0190x over previous
import functools

import jax
import jax.numpy as jnp
from jax import lax
from jax.experimental import pallas as pl
from jax.experimental.pallas import tpu as pltpu
from jax.experimental.pallas import tpu_sc as plsc

_K = 16
_RB = 512
_RC = 512
_RF = 512
_SC_CORES = 2
_SC_SUBCORES = 16
_GCHUNK = 1024


def _mish(v):
    sp = jnp.maximum(v, 0.0) + jnp.log1p(jnp.exp(-jnp.abs(v)))
    return v * jnp.tanh(sp)


def _topk_body(ps_ref, pos_ref, idx_ref, ds_ref):
    r = ps_ref.shape[1]
    n = pos_ref.shape[2]
    xb = ps_ref[0, :, 0:1]
    yb = ps_ref[0, :, 1:2]
    zb = ps_ref[0, :, 2:3]
    xf = pos_ref[0, 0:1, :]
    yf = pos_ref[0, 1:2, :]
    zf = pos_ref[0, 2:3, :]
    sqb = xb * xb + yb * yb + zb * zb
    sqf = xf * xf + yf * yf + zf * zf
    dot = jnp.dot(ps_ref[0].astype(jnp.bfloat16),
                  pos_ref[0].astype(jnp.bfloat16),
                  preferred_element_type=jnp.float32)
    d = sqb + sqf - 2.0 * dot
    cols = lax.broadcasted_iota(jnp.int32, (r, n), 1)
    cur = d
    picks = []
    for _ in range(_K):
        j = jnp.argmin(cur, axis=1, keepdims=True).astype(jnp.int32)
        picks.append(j)
        cur = jnp.where(cols == j, jnp.float32(jnp.inf), cur)
    idx_ref[0] = jnp.concatenate(picks, axis=1)
    ds_ref[0] = jnp.sum(jnp.where(jnp.isinf(cur), d, 0.0), axis=1,
                        keepdims=True)


def _topk(pos):
    b, _, n = pos.shape
    ps = jnp.transpose(pos, (0, 2, 1))
    rb = min(_RB, n)
    nb = n // rb
    idx, ds = pl.pallas_call(
        _topk_body,
        grid=(b, nb),
        in_specs=[
            pl.BlockSpec((1, rb, 3), lambda bi, i: (bi, i, 0)),
            pl.BlockSpec((1, 3, n), lambda bi, i: (bi, 0, 0)),
        ],
        out_specs=[
            pl.BlockSpec((1, rb, _K), lambda bi, i: (bi, i, 0)),
            pl.BlockSpec((1, rb, 1), lambda bi, i: (bi, i, 0)),
        ],
        out_shape=[
            jax.ShapeDtypeStruct((b, n, _K), jnp.int32),
            jax.ShapeDtypeStruct((b, n, 1), jnp.float32),
        ],
    )(ps, pos)
    return idx, ds[..., 0]


def _gather_sc(table, idx_flat):
    tot = idx_flat.shape[0]
    c = table.shape[1]
    nw = _SC_CORES * _SC_SUBCORES
    per_w = tot // nw
    chunk = min(_GCHUNK, per_w)
    nch = per_w // chunk
    mesh = plsc.VectorSubcoreMesh(
        core_axis_name="c", subcore_axis_name="s",
        num_cores=_SC_CORES, num_subcores=_SC_SUBCORES)

    @functools.partial(
        pl.kernel,
        out_type=jax.ShapeDtypeStruct((tot, c), jnp.float32),
        mesh=mesh,
        scratch_types=[
            pltpu.VMEM((chunk,), jnp.int32),
            pltpu.VMEM((chunk, c), jnp.float32),
            pltpu.SemaphoreType.DMA,
        ],
        compiler_params=pltpu.CompilerParams(use_tc_tiling_on_sc=False),
    )
    def k(table_hbm, idx_hbm, out_hbm, idx_v, rows_v, sem):
        wid = lax.axis_index("s") * _SC_CORES + lax.axis_index("c")
        base = wid * per_w
        for i in range(nch):
            off = base + i * chunk
            pltpu.sync_copy(idx_hbm.at[pl.ds(off, chunk)], idx_v)
            pltpu.async_copy(table_hbm.at[idx_v], rows_v, sem).wait()
            pltpu.sync_copy(rows_v, out_hbm.at[pl.ds(off, chunk)])

    return k(table, idx_flat)


def _conv1_body(nbr_ref, xt_ref, w1a_ref, w1b_ref, b1_ref,
                h1_ref, s1_ref, q1_ref):
    r = xt_ref.shape[0]
    c = xt_ref.shape[1]
    nbr = nbr_ref[...].reshape(r, _K, c)
    xi = xt_ref[...]
    en = (nbr - xi[:, None, :]).reshape(r * _K, c)
    ha = jnp.dot(en, w1a_ref[...], preferred_element_type=jnp.float32)
    hb = jnp.dot(xi, w1b_ref[...], preferred_element_type=jnp.float32)
    h = (ha.reshape(r, _K, -1) + hb[:, None, :]).reshape(r * _K, -1)
    h = h + b1_ref[...]
    h1_ref[...] = h
    s = jnp.sum(h, axis=0, keepdims=True)
    q = jnp.sum(h * h, axis=0, keepdims=True)

    @pl.when(pl.program_id(0) == 0)
    def _():
        s1_ref[...] = jnp.zeros_like(s1_ref)
        q1_ref[...] = jnp.zeros_like(q1_ref)

    s1_ref[...] += s
    q1_ref[...] += q


def _conv1(nbr, xt, w1, b1):
    rows, c = xt.shape
    dmid = w1.shape[0]
    rc = min(_RC, rows)
    ng = rows // rc
    w1a = jnp.transpose(w1[:, :c])
    w1b = jnp.transpose(w1[:, c:])
    return pl.pallas_call(
        _conv1_body,
        grid=(ng,),
        in_specs=[
            pl.BlockSpec((rc * _K, c), lambda g: (g, 0)),
            pl.BlockSpec((rc, c), lambda g: (g, 0)),
            pl.BlockSpec((c, dmid), lambda g: (0, 0)),
            pl.BlockSpec((c, dmid), lambda g: (0, 0)),
            pl.BlockSpec((1, dmid), lambda g: (0, 0)),
        ],
        out_specs=[
            pl.BlockSpec((rc * _K, dmid), lambda g: (g, 0)),
            pl.BlockSpec((1, dmid), lambda g: (0, 0)),
            pl.BlockSpec((1, dmid), lambda g: (0, 0)),
        ],
        out_shape=[
            jax.ShapeDtypeStruct((rows * _K, dmid), jnp.float32),
            jax.ShapeDtypeStruct((1, dmid), jnp.float32),
            jax.ShapeDtypeStruct((1, dmid), jnp.float32),
        ],
    )(nbr, xt, w1a, w1b, b1.reshape(1, -1))


def _conv2_body(h1_ref, sc_ref, sh_ref, w2_ref, b2_ref,
                h2_ref, s2_ref, q2_ref):
    t = h1_ref[...] * sc_ref[...] + sh_ref[...]
    g = _mish(t)
    h = jnp.dot(g, w2_ref[...], preferred_element_type=jnp.float32) + b2_ref[...]
    h2_ref[...] = h
    s = jnp.sum(h, axis=0, keepdims=True)
    q = jnp.sum(h * h, axis=0, keepdims=True)

    @pl.when(pl.program_id(0) == 0)
    def _():
        s2_ref[...] = jnp.zeros_like(s2_ref)
        q2_ref[...] = jnp.zeros_like(q2_ref)

    s2_ref[...] += s
    q2_ref[...] += q


def _conv2(h1, sc1, sh1, w2, b2):
    rows_k, dmid = h1.shape
    dout = w2.shape[0]
    blk = min(_RC * _K, rows_k)
    ng = rows_k // blk
    w2t = jnp.transpose(w2)
    return pl.pallas_call(
        _conv2_body,
        grid=(ng,),
        in_specs=[
            pl.BlockSpec((blk, dmid), lambda g: (g, 0)),
            pl.BlockSpec((1, dmid), lambda g: (0, 0)),
            pl.BlockSpec((1, dmid), lambda g: (0, 0)),
            pl.BlockSpec((dmid, dout), lambda g: (0, 0)),
            pl.BlockSpec((1, dout), lambda g: (0, 0)),
        ],
        out_specs=[
            pl.BlockSpec((blk, dout), lambda g: (g, 0)),
            pl.BlockSpec((1, dout), lambda g: (0, 0)),
            pl.BlockSpec((1, dout), lambda g: (0, 0)),
        ],
        out_shape=[
            jax.ShapeDtypeStruct((rows_k, dout), jnp.float32),
            jax.ShapeDtypeStruct((1, dout), jnp.float32),
            jax.ShapeDtypeStruct((1, dout), jnp.float32),
        ],
    )(h1, sc1, sh1, w2t, b2.reshape(1, -1))


def _final_body(h2_ref, sc_ref, sh_ref, r_ref, s3_ref):
    r = r_ref.shape[0]
    t = h2_ref[...] * sc_ref[...] + sh_ref[...]
    m = _mish(t)
    r_ref[...] = jnp.max(m.reshape(r, _K, -1), axis=1)
    s = jnp.sum(m, axis=0, keepdims=True)

    @pl.when(pl.program_id(1) == 0)
    def _():
        s3_ref[...] = jnp.zeros_like(s3_ref)

    s3_ref[0] += s


def _finalize(h2, sc2, sh2, b, n):
    rows_k, dout = h2.shape
    rc = min(_RC, n)
    nb = n // rc
    return pl.pallas_call(
        _final_body,
        grid=(b, nb),
        in_specs=[
            pl.BlockSpec((rc * _K, dout), lambda bi, i: (bi * nb + i, 0)),
            pl.BlockSpec((1, dout), lambda bi, i: (0, 0)),
            pl.BlockSpec((1, dout), lambda bi, i: (0, 0)),
        ],
        out_specs=[
            pl.BlockSpec((rc, dout), lambda bi, i: (bi * nb + i, 0)),
            pl.BlockSpec((1, 1, dout), lambda bi, i: (bi, 0, 0)),
        ],
        out_shape=[
            jax.ShapeDtypeStruct((b * n, dout), jnp.float32),
            jax.ShapeDtypeStruct((b, 1, dout), jnp.float32),
        ],
    )(h2, sc2, sh2)


def _scale_body(cnt_inv, r_ref, s3_ref, w1_ref, b1_ref, w2t_ref, b2_ref,
                out_ref):
    sm = s3_ref[0] * cnt_inv
    z = jnp.sum(w1_ref[...] * sm, axis=1, keepdims=True)
    z = jnp.maximum(z + b1_ref[...], 0.0)
    e = jnp.sum(w2t_ref[...] * z, axis=0, keepdims=True)
    e = 1.0 / (1.0 + jnp.exp(-(e + b2_ref[...])))
    h = r_ref[...] * e
    out_ref[0] = jnp.transpose(h)


def _scale_out(r, s3, se_w1, se_b1, se_w2, se_b2, b, n, cnt):
    dout = r.shape[1]
    dse = se_w1.shape[0]
    rf = min(_RF, n)
    nb = n // rf
    body = functools.partial(_scale_body, 1.0 / cnt)
    return pl.pallas_call(
        body,
        grid=(b, nb),
        in_specs=[
            pl.BlockSpec((rf, dout), lambda bi, i: (bi * nb + i, 0)),
            pl.BlockSpec((1, 1, dout), lambda bi, i: (bi, 0, 0)),
            pl.BlockSpec((dse, dout), lambda bi, i: (0, 0)),
            pl.BlockSpec((dse, 1), lambda bi, i: (0, 0)),
            pl.BlockSpec((dse, dout), lambda bi, i: (0, 0)),
            pl.BlockSpec((1, dout), lambda bi, i: (0, 0)),
        ],
        out_specs=pl.BlockSpec((1, dout, rf), lambda bi, i: (bi, 0, i)),
        out_shape=jax.ShapeDtypeStruct((b, dout, n), jnp.float32),
    )(r, s3, se_w1, se_b1.reshape(-1, 1), jnp.transpose(se_w2), se_b2.reshape(1, -1))


def kernel(x, pos, conv1_W, conv1_b, bn1_g, bn1_b, conv2_W, conv2_b,
           bn2_g, bn2_b, se_W1, se_b1, se_W2, se_b2):
    b, c, n = x.shape
    cnt = b * n * _K

    idx, dist_sum = _topk(pos)

    xt = jnp.transpose(x, (0, 2, 1)).reshape(b * n, c)
    offs = (jnp.arange(b, dtype=jnp.int32) * n)[:, None, None]
    idx_flat = (idx + offs).reshape(-1)
    nbr = _gather_sc(xt, idx_flat)

    h1, s1, q1 = _conv1(nbr, xt, conv1_W, conv1_b)
    mu1 = s1 / cnt
    var1 = q1 / cnt - mu1 * mu1
    sc1 = bn1_g.reshape(1, -1) / jnp.sqrt(var1 + 1e-5)
    sh1 = bn1_b.reshape(1, -1) - mu1 * sc1

    h2, s2, q2 = _conv2(h1, sc1, sh1, conv2_W, conv2_b)
    mu2 = s2 / cnt
    var2 = q2 / cnt - mu2 * mu2
    sc2 = bn2_g.reshape(1, -1) / jnp.sqrt(var2 + 1e-5)
    sh2 = bn2_b.reshape(1, -1) - mu2 * sc2

    r, s3 = _finalize(h2, sc2, sh2, b, n)
    residual = _scale_out(r, s3, se_W1, se_b1, se_W2, se_b2, b, n, n * _K)
    return residual, dist_sum
